# baseline (speedup 1.0000x reference)
import functools

import jax
import jax.numpy as jnp
from jax import lax
from jax.experimental import pallas as pl
from jax.experimental.pallas import tpu as pltpu
from jax.experimental.pallas import tpu_sc as plsc

_D = 256
_V = 16
_NC, _NS = 2, 16
_NW = _NC * _NS
_CH = 64
_NCHUNK = 16
_NSTRM = 5
_NCOMP = _NCHUNK - _NSTRM
_NB_C = 2
_NB_S = 3
_L = 16


def _emb_body(bpw, ids_hbm, table_hbm, table2_hbm, out_hbm,
              idx_v, table_v, rows_c, rows_s, csem, gsem, ssem):
    wid = lax.axis_index("s") * _NC + lax.axis_index("c")
    base = wid * bpw

    pltpu.sync_copy(table_hbm, table_v)
    pltpu.sync_copy(ids_hbm.at[pl.ds(base, bpw)], idx_v.at[pl.ds(0, bpw)])

    lanes = lax.iota(jnp.int32, _L)
    cols = [lanes + j * _L for j in range(_D // _L)]
    zsplat = jnp.zeros((_L, 1), jnp.int32)
    bcast_dims = lax.GatherDimensionNumbers(
        offset_dims=(), collapsed_slice_dims=(0,), start_index_map=(0,))

    def build(t, b):
        @plsc.parallel_loop(0, _CH, 1, unroll=4)
        def _row(i):
            idvec = idx_v[pl.ds(t * _CH + i, _L)]
            rbase = lax.gather(idvec, zsplat, bcast_dims, (1,),
                               mode=lax.GatherScatterMode.PROMISE_IN_BOUNDS)
            rbase = rbase * _D
            for j in range(_D // _L):
                vec = plsc.load_gather(table_v, [rbase + cols[j]])
                rows_c[b, i, pl.ds(j * _L, _L)] = vec

    def store_from(buf_ref, sem, t):
        return pltpu.async_copy(
            buf_ref, out_hbm.at[pl.ds(base + t * _CH, _CH)], sem)

    def gather(t, sb):
        idx = idx_v.at[pl.ds(t * _CH, _CH)]
        return pltpu.async_copy(table2_hbm.at[idx], rows_s.at[sb],
                                gsem.at[sb])

    strm0 = _NCOMP
    g = {0: gather(strm0 + 0, 0),
         1: gather(strm0 + 1, 1),
         2: gather(strm0 + 2, 2)}
    cst = {}
    sst = {}

    def fire_compute(t):
        b = t % _NB_C
        if t >= _NB_C:
            cst[t - _NB_C].wait()
        build(t, b)
        cst[t] = store_from(rows_c.at[b], csem.at[b], t)

    def service(k):
        sb = k % _NB_S
        g[k].wait()
        sst[k] = store_from(rows_s.at[sb], ssem.at[sb], strm0 + k)
        nk = k + 2
        if _NB_S <= nk < _NSTRM:
            sst[nk - _NB_S].wait()
            g[nk] = gather(strm0 + nk, nk % _NB_S)

    fire_compute(0)
    fire_compute(1)
    service(0)
    fire_compute(2)
    fire_compute(3)
    service(1)
    fire_compute(4)
    fire_compute(5)
    service(2)
    fire_compute(6)
    fire_compute(7)
    service(3)
    fire_compute(8)
    fire_compute(9)
    service(4)
    fire_compute(10)

    cst[_NCOMP - 2].wait()
    cst[_NCOMP - 1].wait()
    for k in range(_NSTRM - _NB_S, _NSTRM):
        sst[k].wait()


@functools.partial(jax.jit, static_argnums=(3,))
def _emb(flat_ids, flat_table, table2d, n):
    bpw = n // _NW
    grid_kernel = functools.partial(
        pl.kernel,
        out_type=jax.ShapeDtypeStruct((n, _D), jnp.float32),
        mesh=plsc.VectorSubcoreMesh(core_axis_name="c", subcore_axis_name="s"),
        compiler_params=pltpu.CompilerParams(needs_layout_passes=False),
        scratch_types=[
            pltpu.VMEM((bpw + _L,), jnp.int32),
            pltpu.VMEM((_V * _D,), jnp.float32),
            pltpu.VMEM((_NB_C, _CH, _D), jnp.float32),
            pltpu.VMEM((_NB_S, _CH, _D), jnp.float32),
            pltpu.SemaphoreType.DMA((_NB_C,)),
            pltpu.SemaphoreType.DMA((_NB_S,)),
            pltpu.SemaphoreType.DMA((_NB_S,)),
        ],
    )
    return grid_kernel(functools.partial(_emb_body, bpw))(
        flat_ids, flat_table, table2d)


def kernel(input_ids, table):
    n = input_ids.size
    flat = input_ids.reshape((n,))
    out = _emb(flat, table.reshape((-1,)), table, n)
    return out.reshape(input_ids.shape + (table.shape[1],))


# speedup vs baseline: 1.0074x; 1.0074x over previous
import functools

import jax
import jax.numpy as jnp
from jax import lax
from jax.experimental import pallas as pl
from jax.experimental.pallas import tpu as pltpu
from jax.experimental.pallas import tpu_sc as plsc

_D = 256
_V = 16
_NC, _NS = 2, 16
_NW = _NC * _NS
_CH = 128
_NBUF = 2
_L = 16


def _emb_body(bpw, ids_hbm, table_hbm, out_hbm, idx_v, table_v, rows_v, ssem):
    nchunk = bpw // _CH
    wid = lax.axis_index("s") * _NC + lax.axis_index("c")
    base = wid * bpw

    pltpu.sync_copy(table_hbm, table_v)
    pltpu.sync_copy(ids_hbm.at[pl.ds(base, bpw)], idx_v.at[pl.ds(0, bpw)])

    lanes = lax.iota(jnp.int32, _L)
    cols = [lanes + j * _L for j in range(_D // _L)]
    zsplat = jnp.zeros((_L, 1), jnp.int32)
    bcast_dims = lax.GatherDimensionNumbers(
        offset_dims=(), collapsed_slice_dims=(0,), start_index_map=(0,))

    def lane0_bcast(vec):
        return lax.gather(vec, zsplat, bcast_dims, (1,),
                          mode=lax.GatherScatterMode.PROMISE_IN_BOUNDS)

    def build(t, b):
        @plsc.parallel_loop(0, _CH // 2, 1, unroll=2)
        def _rowpair(p):
            i0 = p * 2
            rb0 = lane0_bcast(idx_v[pl.ds(t * _CH + i0, _L)]) * _D
            rb1 = lane0_bcast(idx_v[pl.ds(t * _CH + i0 + 1, _L)]) * _D
            for j in range(_D // _L):
                v0 = plsc.load_gather(table_v, [rb0 + cols[j]])
                v1 = plsc.load_gather(table_v, [rb1 + cols[j]])
                rows_v[b, pl.ds(i0 * _D + j * _L, _L)] = v0
                rows_v[b, pl.ds((i0 + 1) * _D + j * _L, _L)] = v1

    _CHD = _CH * _D

    def fire(t, b):
        build(t, b)
        return pltpu.async_copy(
            rows_v.at[b], out_hbm.at[pl.ds((base + t * _CH) * _D, _CHD)],
            ssem.at[b])

    def drain(b):
        pltpu.make_async_copy(
            rows_v.at[b], out_hbm.at[pl.ds(base * _D, _CHD)],
            ssem.at[b]).wait()

    for b in range(_NBUF):
        fire(b, b)

    def step(k, carry):
        for b in range(_NBUF):
            drain(b)
            fire(k * _NBUF + b, b)
        return carry

    lax.fori_loop(1, nchunk // _NBUF, step, 0)
    for b in range(_NBUF):
        drain(b)


@functools.partial(jax.jit, static_argnums=(2,))
def _emb(flat_ids, flat_table, n):
    bpw = n // _NW
    grid_kernel = functools.partial(
        pl.kernel,
        out_type=jax.ShapeDtypeStruct((n * _D,), jnp.float32),
        mesh=plsc.VectorSubcoreMesh(core_axis_name="c", subcore_axis_name="s"),
        compiler_params=pltpu.CompilerParams(needs_layout_passes=False),
        scratch_types=[
            pltpu.VMEM((bpw + _L,), jnp.int32),
            pltpu.VMEM((_V * _D,), jnp.float32),
            pltpu.VMEM((_NBUF, _CH * _D), jnp.float32),
            pltpu.SemaphoreType.DMA((_NBUF,)),
        ],
    )
    return grid_kernel(functools.partial(_emb_body, bpw))(flat_ids, flat_table)


def kernel(input_ids, table):
    n = input_ids.size
    flat = input_ids.reshape((n,))
    out = _emb(flat, table.reshape((-1,)), n)
    return out.reshape(input_ids.shape + (table.shape[1],))


# speedup vs baseline: 1.0890x; 1.0810x over previous
import functools

import jax
import jax.numpy as jnp
from jax import lax
from jax.experimental import pallas as pl
from jax.experimental.pallas import tpu as pltpu
from jax.experimental.pallas import tpu_sc as plsc

_D = 256
_V = 16
_NC, _NS = 2, 16
_NW = _NC * _NS
_CH = 128
_NBUF = 2
_L = 16


def _emb_body(bpw, ids_hbm, table_hbm, out_hbm, idx_v, table_v, rows_v, ssem):
    nchunk = bpw // _CH
    wid = lax.axis_index("s") * _NC + lax.axis_index("c")
    base = wid * bpw

    pltpu.sync_copy(table_hbm, table_v)
    pltpu.sync_copy(ids_hbm.at[pl.ds(base, bpw)], idx_v.at[pl.ds(0, bpw)])

    lanes = lax.iota(jnp.int32, _L)
    cols = [lanes + j * _L for j in range(_D // _L)]
    zsplat = jnp.zeros((_L, 1), jnp.int32)
    bcast_dims = lax.GatherDimensionNumbers(
        offset_dims=(), collapsed_slice_dims=(0,), start_index_map=(0,))

    def build(t, b):
        @plsc.parallel_loop(0, _CH, 1, unroll=4)
        def _row(i):
            idvec = idx_v[pl.ds(t * _CH + i, _L)]
            rbase = lax.gather(idvec, zsplat, bcast_dims, (1,),
                               mode=lax.GatherScatterMode.PROMISE_IN_BOUNDS)
            rbase = rbase * _D
            for j in range(_D // _L):
                vec = plsc.load_gather(table_v, [rbase + cols[j]])
                rows_v[b, pl.ds(i * _D + j * _L, _L)] = vec

    _CHD = _CH * _D

    def fire(t, b):
        build(t, b)
        return pltpu.async_copy(
            rows_v.at[b], out_hbm.at[pl.ds((base + t * _CH) * _D, _CHD)],
            ssem.at[b])

    def drain(b):
        pltpu.make_async_copy(
            rows_v.at[b], out_hbm.at[pl.ds(base * _D, _CHD)],
            ssem.at[b]).wait()

    for b in range(_NBUF):
        fire(b, b)

    def step(k, carry):
        for b in range(_NBUF):
            drain(b)
            fire(k * _NBUF + b, b)
        return carry

    lax.fori_loop(1, nchunk // _NBUF, step, 0)
    for b in range(_NBUF):
        drain(b)


@functools.partial(jax.jit, static_argnums=(2,))
def _emb(flat_ids, flat_table, n):
    bpw = n // _NW
    grid_kernel = functools.partial(
        pl.kernel,
        out_type=jax.ShapeDtypeStruct((n * _D,), jnp.float32),
        mesh=plsc.VectorSubcoreMesh(core_axis_name="c", subcore_axis_name="s"),
        compiler_params=pltpu.CompilerParams(needs_layout_passes=False),
        scratch_types=[
            pltpu.VMEM((bpw + _L,), jnp.int32),
            pltpu.VMEM((_V * _D,), jnp.float32),
            pltpu.VMEM((_NBUF, _CH * _D), jnp.float32),
            pltpu.SemaphoreType.DMA((_NBUF,)),
        ],
    )
    return grid_kernel(functools.partial(_emb_body, bpw))(flat_ids, flat_table)


def kernel(input_ids, table):
    n = input_ids.size
    flat = input_ids.reshape((n,))
    out = _emb(flat, table.reshape((-1,)), n)
    return out.reshape(input_ids.shape + (table.shape[1],))
